# baseline (speedup 1.0000x reference)
import functools

import jax
import jax.numpy as jnp
from jax import lax
from jax.experimental import pallas as pl
from jax.experimental.pallas import tpu as pltpu
from jax.experimental.pallas import tpu_sc as plsc

B = 64
C = 384
H = 32
WD = 32
HW = 1024
E = 16
NSUB = 32
BATCHES_PER_SUB = B // NSUB
SLAB = WD * C
NCHUNK = BATCHES_PER_SUB * H
GROUPS = BATCHES_PER_SUB * C // 16
GPB = C // 16


def _bf16_rne(v):
    u = lax.bitcast_convert_type(v, jnp.uint32)
    u = (u + jnp.uint32(0x7FFF) + ((u >> jnp.uint32(16)) & jnp.uint32(1))) \
        & jnp.uint32(0xFFFF0000)
    return lax.bitcast_convert_type(u, jnp.float32)


def _recip_nw(d):
    r = 1.0 / d
    r = r * (2.0 - d * r)
    r = r * (2.0 - d * r)
    return r


def _rsqrt_nw(v):
    i = lax.bitcast_convert_type(v, jnp.int32)
    i = jnp.int32(0x5F3759DF) - (i >> 1)
    y = lax.bitcast_convert_type(i, jnp.float32)
    for _ in range(4):
        y = y * (1.5 - 0.5 * v * y * y)
    return y


def _sc_router(x_flat, wt_flat):
    mesh = plsc.VectorSubcoreMesh(core_axis_name="c", subcore_axis_name="s")
    f32 = jnp.float32
    i32 = jnp.int32

    out_type = (
        jax.ShapeDtypeStruct((B * E,), f32),
        jax.ShapeDtypeStruct((B * E,), f32),
        jax.ShapeDtypeStruct((B * E,), f32),
        jax.ShapeDtypeStruct((B * E,), i32),
        jax.ShapeDtypeStruct((NSUB * E,), f32),
    )
    scratch = [
        pltpu.VMEM((SLAB,), f32),
        pltpu.VMEM((SLAB,), f32),
        pltpu.VMEM((2 * C * E,), f32),
        pltpu.VMEM((GROUPS * 16,), f32),
        pltpu.VMEM((GROUPS * 16,), f32),
        pltpu.VMEM((GROUPS * 16,), f32),
        pltpu.VMEM((GROUPS * 16,), f32),
        pltpu.VMEM((BATCHES_PER_SUB * E,), f32),
        pltpu.VMEM((BATCHES_PER_SUB * E,), f32),
        pltpu.VMEM((BATCHES_PER_SUB * E,), f32),
        pltpu.VMEM((BATCHES_PER_SUB * E,), i32),
        pltpu.VMEM((E,), f32),
        pltpu.SemaphoreType.DMA,
        pltpu.SemaphoreType.DMA,
    ]

    @functools.partial(pl.kernel, out_type=out_type, mesh=mesh,
                       scratch_types=scratch,
                       compiler_params=pltpu.CompilerParams(
                           needs_layout_passes=False))
    def sck(x_hbm, wt_hbm, probs_o, logits_o, wts_o, idx_o, usage_o,
            buf0, buf1, wt_vm, mean_vm, std_vm, accs, accq,
            ob_probs, ob_logits, ob_wts, ob_idx, ob_usage,
            sem0, sem1):
        wid = lax.axis_index("s") * 2 + lax.axis_index("c")
        base_elem = wid * NCHUNK * SLAB

        pltpu.sync_copy(wt_hbm, wt_vm)

        iota = lax.iota(i32, 16)
        zero = jnp.zeros((16,), f32)

        def start(k, buf, sem):
            src = x_hbm.at[pl.ds(base_elem + k * SLAB, SLAB)]
            pltpu.async_copy(src, buf, sem)

        def wait(k, buf, sem):
            src = x_hbm.at[pl.ds(base_elem + k * SLAB, SLAB)]
            pltpu.make_async_copy(src, buf, sem).wait()

        def zbody(g, carry):
            accs[pl.ds(g * 16, 16)] = zero
            accq[pl.ds(g * 16, 16)] = zero
            return carry

        lax.fori_loop(0, GROUPS, zbody, 0)

        def reduce_pair(kc):
            wait(kc, buf0, sem0)
            wait(kc + 1, buf1, sem1)
            roff = (kc // H) * C

            def cgbody(cg):
                cb = (cg // 8) * 1024 + (cg % 8) * 16
                off = roff + cg * 16
                s_ = [accs[pl.ds(off, 16)], zero, zero, zero]
                q_ = [accq[pl.ds(off, 16)], zero, zero, zero]
                n = 0
                for bufx in (buf0, buf1):
                    for wt in range(WD // 8):
                        for ws in range(8):
                            v = bufx[pl.ds(cb + wt * 3072 + ws * 128, 16)]
                            s_[n % 4] = s_[n % 4] + v
                            q_[n % 4] = q_[n % 4] + v * v
                            n += 1
                accs[pl.ds(off, 16)] = (s_[0] + s_[1]) + (s_[2] + s_[3])
                accq[pl.ds(off, 16)] = (q_[0] + q_[1]) + (q_[2] + q_[3])

            plsc.parallel_loop(0, GPB, 1, unroll=2)(cgbody)

            @pl.when(kc + 2 < NCHUNK)
            def _():
                start(kc + 2, buf0, sem0)
                start(kc + 3, buf1, sem1)

        start(0, buf0, sem0)
        start(1, buf1, sem1)

        def chunk_pair(i, carry):
            reduce_pair(2 * i)
            return carry

        lax.fori_loop(0, NCHUNK // 2, chunk_pair, 0)

        def statbody(g, carry):
            s = accs[pl.ds(g * 16, 16)]
            s2 = accq[pl.ds(g * 16, 16)]
            mean = s * (1.0 / HW)
            var = (s2 - s * s * (1.0 / HW)) * (1.0 / (HW - 1))
            vc = jnp.maximum(var, 1e-30)
            std = vc * _rsqrt_nw(vc)
            mean_vm[pl.ds(g * 16, 16)] = _bf16_rne(mean)
            std_vm[pl.ds(g * 16, 16)] = _bf16_rne(std)
            return carry

        lax.fori_loop(0, GROUPS, statbody, 0)

        usage = jnp.zeros((16,), f32)
        for bl in range(BATCHES_PER_SUB):
            def p2body(kg, lins):
                base = (bl * GPB + kg) * 16
                mv = mean_vm[pl.ds(base, 16)]
                sv = std_vm[pl.ds(base, 16)]
                lins = list(lins)
                for l in range(16):
                    cc = kg * 16 + l
                    wm = wt_vm[pl.ds(cc * E, E)]
                    ws = wt_vm[pl.ds((C + cc) * E, E)]
                    lins[l % 4] = lins[l % 4] + mv[l] * wm + sv[l] * ws
                return tuple(lins)

            lins = plsc.parallel_loop(0, GPB, 1, unroll=2,
                                      carry=(zero,) * 4)(p2body)
            lin = (lins[0] + lins[1]) + (lins[2] + lins[3])

            m1 = jnp.max(lin)
            e1 = jnp.exp(lin - m1)
            p1 = e1 * _recip_nw(zero + jnp.sum(e1))
            lg = jnp.clip(p1, -30.0, 30.0)
            m2 = jnp.max(lg)
            e2 = jnp.exp(lg - m2)
            p2 = e2 * _recip_nw(zero + jnp.sum(e2))
            v1 = jnp.max(p2)
            i1 = jnp.min(jnp.where(p2 == v1, iota, E))
            neg = jnp.where(iota == i1, -1e30, p2)
            v2 = jnp.max(neg)
            i2 = jnp.min(jnp.where(neg == v2, iota, E))
            den = v1 + v2 + 1e-6
            wnum = jnp.where(iota == 0, v1, jnp.where(iota == 1, v2, 0.0))
            wvec = wnum * _recip_nw(zero + den)

            ob_probs[pl.ds(bl * E, E)] = p2
            ob_logits[pl.ds(bl * E, E)] = lg
            ob_wts[pl.ds(bl * E, E)] = wvec
            ob_idx[pl.ds(bl * E, E)] = jnp.where(iota == 0, i1,
                                                 jnp.where(iota == 1, i2, 0))
            usage = usage + jnp.where(iota == i1, 1.0, 0.0) \
                          + jnp.where(iota == i2, 1.0, 0.0)

        ob_usage[...] = usage
        obase = wid * BATCHES_PER_SUB * E
        nout = BATCHES_PER_SUB * E
        pltpu.sync_copy(ob_probs, probs_o.at[pl.ds(obase, nout)])
        pltpu.sync_copy(ob_logits, logits_o.at[pl.ds(obase, nout)])
        pltpu.sync_copy(ob_wts, wts_o.at[pl.ds(obase, nout)])
        pltpu.sync_copy(ob_idx, idx_o.at[pl.ds(obase, nout)])
        pltpu.sync_copy(ob_usage, usage_o.at[pl.ds(wid * E, E)])

    return sck(x_flat, wt_flat)


@jax.jit
def kernel(x, W):
    xp = x.transpose(0, 2, 3, 1).reshape(B, H, WD // 8, 8, C // 128, 128)
    x_flat = xp.transpose(0, 1, 2, 4, 3, 5).reshape(-1)
    wt_flat = W.T.astype(jnp.bfloat16).astype(jnp.float32).reshape(-1)
    probs, logits, wts, idxo, usage = _sc_router(x_flat, wt_flat)
    probs = probs.reshape(B, E)
    logits = logits.reshape(B, E)
    ti = idxo.reshape(B, E)[:, :2]
    routing_weights = wts.reshape(B, E)[:, :2].reshape(B, 2, 1, 1)
    routing_indices = ti.reshape(B, 2, 1, 1)
    expert_usage = jnp.sum(usage.reshape(NSUB, E), axis=0) * (1.0 / (B * 2))
    return (routing_weights, routing_indices, probs, logits, ti, expert_usage)


# speedup vs baseline: 1.3612x; 1.3612x over previous
import functools

import jax
import jax.numpy as jnp
from jax import lax
from jax.experimental import pallas as pl
from jax.experimental.pallas import tpu as pltpu
from jax.experimental.pallas import tpu_sc as plsc

B = 64
C = 384
H = 32
WD = 32
HW = 1024
E = 16
NSUB = 32
BATCHES_PER_SUB = B // NSUB
SLAB = WD * C
NCHUNK = BATCHES_PER_SUB * H
GROUPS = BATCHES_PER_SUB * C // 16
GPB = C // 16


def _bf16_rne(v):
    u = lax.bitcast_convert_type(v, jnp.uint32)
    u = (u + jnp.uint32(0x7FFF) + ((u >> jnp.uint32(16)) & jnp.uint32(1))) \
        & jnp.uint32(0xFFFF0000)
    return lax.bitcast_convert_type(u, jnp.float32)


def _recip_nw(d):
    r = 1.0 / d
    r = r * (2.0 - d * r)
    r = r * (2.0 - d * r)
    return r


def _rsqrt_nw(v):
    i = lax.bitcast_convert_type(v, jnp.int32)
    i = jnp.int32(0x5F3759DF) - (i >> 1)
    y = lax.bitcast_convert_type(i, jnp.float32)
    for _ in range(4):
        y = y * (1.5 - 0.5 * v * y * y)
    return y


def _sc_router(x_flat, wt_flat):
    mesh = plsc.VectorSubcoreMesh(core_axis_name="c", subcore_axis_name="s")
    f32 = jnp.float32
    i32 = jnp.int32

    out_type = (
        jax.ShapeDtypeStruct((B * E,), f32),
        jax.ShapeDtypeStruct((B * E,), f32),
        jax.ShapeDtypeStruct((B * E,), f32),
        jax.ShapeDtypeStruct((B * E,), i32),
        jax.ShapeDtypeStruct((NSUB * E,), f32),
    )
    scratch = [
        pltpu.VMEM((SLAB,), f32),
        pltpu.VMEM((SLAB,), f32),
        pltpu.VMEM((2 * C * E,), f32),
        pltpu.VMEM((GROUPS * 16,), f32),
        pltpu.VMEM((GROUPS * 16,), f32),
        pltpu.VMEM((GROUPS * 16,), f32),
        pltpu.VMEM((GROUPS * 16,), f32),
        pltpu.VMEM((BATCHES_PER_SUB * E,), f32),
        pltpu.VMEM((BATCHES_PER_SUB * E,), f32),
        pltpu.VMEM((BATCHES_PER_SUB * E,), f32),
        pltpu.VMEM((BATCHES_PER_SUB * E,), i32),
        pltpu.VMEM((E,), f32),
        pltpu.SemaphoreType.DMA,
        pltpu.SemaphoreType.DMA,
    ]

    @functools.partial(pl.kernel, out_type=out_type, mesh=mesh,
                       scratch_types=scratch,
                       compiler_params=pltpu.CompilerParams(
                           needs_layout_passes=False))
    def sck(x_hbm, wt_hbm, probs_o, logits_o, wts_o, idx_o, usage_o,
            buf0, buf1, wt_vm, mean_vm, std_vm, accs, accq,
            ob_probs, ob_logits, ob_wts, ob_idx, ob_usage,
            sem0, sem1):
        wid = lax.axis_index("s") * 2 + lax.axis_index("c")
        base_elem = wid * NCHUNK * SLAB

        pltpu.sync_copy(wt_hbm, wt_vm)

        iota = lax.iota(i32, 16)
        zero = jnp.zeros((16,), f32)

        def start(k, buf, sem):
            src = x_hbm.at[pl.ds(base_elem + k * SLAB, SLAB)]
            pltpu.async_copy(src, buf, sem)

        def wait(k, buf, sem):
            src = x_hbm.at[pl.ds(base_elem + k * SLAB, SLAB)]
            pltpu.make_async_copy(src, buf, sem).wait()

        def zbody(g, carry):
            accs[pl.ds(g * 16, 16)] = zero
            accq[pl.ds(g * 16, 16)] = zero
            return carry

        lax.fori_loop(0, GROUPS, zbody, 0)

        def reduce_chunk(kc, buf, sem):
            wait(kc, buf, sem)
            roff = (kc // H) * C

            def cgbody(cg):
                cb = (cg // 8) * 1024 + (cg % 8) * 16
                off = roff + cg * 16
                s_ = [accs[pl.ds(off, 16)], zero, zero, zero]
                q_ = [accq[pl.ds(off, 16)], zero, zero, zero]
                n = 0
                for wt in range(WD // 8):
                    for ws in range(8):
                        v = buf[pl.ds(cb + wt * 3072 + ws * 128, 16)]
                        s_[n % 4] = s_[n % 4] + v
                        q_[n % 4] = q_[n % 4] + v * v
                        n += 1
                accs[pl.ds(off, 16)] = (s_[0] + s_[1]) + (s_[2] + s_[3])
                accq[pl.ds(off, 16)] = (q_[0] + q_[1]) + (q_[2] + q_[3])

            plsc.parallel_loop(0, GPB, 1, unroll=4)(cgbody)

            @pl.when(kc + 2 < NCHUNK)
            def _():
                start(kc + 2, buf, sem)

        start(0, buf0, sem0)
        start(1, buf1, sem1)

        def chunk_pair(i, carry):
            reduce_chunk(2 * i, buf0, sem0)
            reduce_chunk(2 * i + 1, buf1, sem1)
            return carry

        lax.fori_loop(0, NCHUNK // 2, chunk_pair, 0)

        def statbody(g, carry):
            s = accs[pl.ds(g * 16, 16)]
            s2 = accq[pl.ds(g * 16, 16)]
            mean = s * (1.0 / HW)
            var = (s2 - s * s * (1.0 / HW)) * (1.0 / (HW - 1))
            vc = jnp.maximum(var, 1e-30)
            std = vc * _rsqrt_nw(vc)
            mean_vm[pl.ds(g * 16, 16)] = _bf16_rne(mean)
            std_vm[pl.ds(g * 16, 16)] = _bf16_rne(std)
            return carry

        lax.fori_loop(0, GROUPS, statbody, 0)

        usage = jnp.zeros((16,), f32)
        for bl in range(BATCHES_PER_SUB):
            def p2body(kg, lins):
                base = (bl * GPB + kg) * 16
                mv = mean_vm[pl.ds(base, 16)]
                sv = std_vm[pl.ds(base, 16)]
                lins = list(lins)
                for l in range(16):
                    cc = kg * 16 + l
                    wm = wt_vm[pl.ds(cc * E, E)]
                    ws = wt_vm[pl.ds((C + cc) * E, E)]
                    lins[l % 4] = lins[l % 4] + mv[l] * wm + sv[l] * ws
                return tuple(lins)

            lins = plsc.parallel_loop(0, GPB, 1, unroll=2,
                                      carry=(zero,) * 4)(p2body)
            lin = (lins[0] + lins[1]) + (lins[2] + lins[3])

            m1 = jnp.max(lin)
            e1 = jnp.exp(lin - m1)
            p1 = e1 * _recip_nw(zero + jnp.sum(e1))
            lg = jnp.clip(p1, -30.0, 30.0)
            m2 = jnp.max(lg)
            e2 = jnp.exp(lg - m2)
            p2 = e2 * _recip_nw(zero + jnp.sum(e2))
            v1 = jnp.max(p2)
            i1 = jnp.min(jnp.where(p2 == v1, iota, E))
            neg = jnp.where(iota == i1, -1e30, p2)
            v2 = jnp.max(neg)
            i2 = jnp.min(jnp.where(neg == v2, iota, E))
            den = v1 + v2 + 1e-6
            wnum = jnp.where(iota == 0, v1, jnp.where(iota == 1, v2, 0.0))
            wvec = wnum * _recip_nw(zero + den)

            ob_probs[pl.ds(bl * E, E)] = p2
            ob_logits[pl.ds(bl * E, E)] = lg
            ob_wts[pl.ds(bl * E, E)] = wvec
            ob_idx[pl.ds(bl * E, E)] = jnp.where(iota == 0, i1,
                                                 jnp.where(iota == 1, i2, 0))
            usage = usage + jnp.where(iota == i1, 1.0, 0.0) \
                          + jnp.where(iota == i2, 1.0, 0.0)

        ob_usage[...] = usage
        obase = wid * BATCHES_PER_SUB * E
        nout = BATCHES_PER_SUB * E
        pltpu.sync_copy(ob_probs, probs_o.at[pl.ds(obase, nout)])
        pltpu.sync_copy(ob_logits, logits_o.at[pl.ds(obase, nout)])
        pltpu.sync_copy(ob_wts, wts_o.at[pl.ds(obase, nout)])
        pltpu.sync_copy(ob_idx, idx_o.at[pl.ds(obase, nout)])
        pltpu.sync_copy(ob_usage, usage_o.at[pl.ds(wid * E, E)])

    return sck(x_flat, wt_flat)


@jax.jit
def kernel(x, W):
    xp = x.transpose(0, 2, 3, 1).reshape(B, H, WD // 8, 8, C // 128, 128)
    x_flat = xp.transpose(0, 1, 2, 4, 3, 5).reshape(-1)
    wt_flat = W.T.astype(jnp.bfloat16).astype(jnp.float32).reshape(-1)
    probs, logits, wts, idxo, usage = _sc_router(x_flat, wt_flat)
    probs = probs.reshape(B, E)
    logits = logits.reshape(B, E)
    ti = idxo.reshape(B, E)[:, :2]
    routing_weights = wts.reshape(B, E)[:, :2].reshape(B, 2, 1, 1)
    routing_indices = ti.reshape(B, 2, 1, 1)
    expert_usage = jnp.sum(usage.reshape(NSUB, E), axis=0) * (1.0 / (B * 2))
    return (routing_weights, routing_indices, probs, logits, ti, expert_usage)


# speedup vs baseline: 1.6343x; 1.2006x over previous
import functools

import jax
import jax.numpy as jnp
from jax import lax
from jax.experimental import pallas as pl
from jax.experimental.pallas import tpu as pltpu
from jax.experimental.pallas import tpu_sc as plsc

B = 64
C = 384
H = 32
WD = 32
HW = 1024
E = 16
NSUB = 32
BATCHES_PER_SUB = B // NSUB
SLAB = WD * C
NCHUNK = BATCHES_PER_SUB * H
GROUPS = BATCHES_PER_SUB * C // 16
GPB = C // 16


def _bf16_rne(v):
    u = lax.bitcast_convert_type(v, jnp.uint32)
    u = (u + jnp.uint32(0x7FFF) + ((u >> jnp.uint32(16)) & jnp.uint32(1))) \
        & jnp.uint32(0xFFFF0000)
    return lax.bitcast_convert_type(u, jnp.float32)


def _recip_nw(d):
    r = 1.0 / d
    r = r * (2.0 - d * r)
    r = r * (2.0 - d * r)
    return r


def _rsqrt_nw(v):
    i = lax.bitcast_convert_type(v, jnp.int32)
    i = jnp.int32(0x5F3759DF) - (i >> 1)
    y = lax.bitcast_convert_type(i, jnp.float32)
    for _ in range(4):
        y = y * (1.5 - 0.5 * v * y * y)
    return y


def _sc_router(x_flat, wt_flat):
    mesh = plsc.VectorSubcoreMesh(core_axis_name="c", subcore_axis_name="s")
    f32 = jnp.float32
    i32 = jnp.int32

    out_type = (
        jax.ShapeDtypeStruct((B * E,), f32),
        jax.ShapeDtypeStruct((B * E,), f32),
        jax.ShapeDtypeStruct((B * E,), f32),
        jax.ShapeDtypeStruct((B * E,), i32),
        jax.ShapeDtypeStruct((NSUB * E,), f32),
    )
    scratch = [
        pltpu.VMEM((SLAB,), f32),
        pltpu.VMEM((SLAB,), f32),
        pltpu.VMEM((SLAB,), f32),
        pltpu.VMEM((2 * C * E,), f32),
        pltpu.VMEM((GROUPS * 16,), f32),
        pltpu.VMEM((GROUPS * 16,), f32),
        pltpu.VMEM((GROUPS * 16,), f32),
        pltpu.VMEM((GROUPS * 16,), f32),
        pltpu.VMEM((BATCHES_PER_SUB * E,), f32),
        pltpu.VMEM((BATCHES_PER_SUB * E,), f32),
        pltpu.VMEM((BATCHES_PER_SUB * E,), f32),
        pltpu.VMEM((BATCHES_PER_SUB * E,), i32),
        pltpu.VMEM((E,), f32),
        pltpu.SemaphoreType.DMA,
        pltpu.SemaphoreType.DMA,
        pltpu.SemaphoreType.DMA,
    ]

    @functools.partial(pl.kernel, out_type=out_type, mesh=mesh,
                       scratch_types=scratch,
                       compiler_params=pltpu.CompilerParams(
                           needs_layout_passes=False))
    def sck(x_hbm, wt_hbm, probs_o, logits_o, wts_o, idx_o, usage_o,
            buf0, buf1, buf2, wt_vm, mean_vm, std_vm, accs, accq,
            ob_probs, ob_logits, ob_wts, ob_idx, ob_usage,
            sem0, sem1, sem2):
        wid = lax.axis_index("s") * 2 + lax.axis_index("c")
        base_elem = wid * NCHUNK * SLAB

        pltpu.sync_copy(wt_hbm, wt_vm)

        iota = lax.iota(i32, 16)
        zero = jnp.zeros((16,), f32)

        def start(k, buf, sem):
            src = x_hbm.at[pl.ds(base_elem + k * SLAB, SLAB)]
            pltpu.async_copy(src, buf, sem)

        def wait(k, buf, sem):
            src = x_hbm.at[pl.ds(base_elem + k * SLAB, SLAB)]
            pltpu.make_async_copy(src, buf, sem).wait()

        def zbody(g, carry):
            accs[pl.ds(g * 16, 16)] = zero
            accq[pl.ds(g * 16, 16)] = zero
            return carry

        lax.fori_loop(0, GROUPS, zbody, 0)

        def reduce_chunk(kc, buf, sem):
            wait(kc, buf, sem)
            roff = (kc // H) * C

            def cgbody(cg):
                cb = (cg // 8) * 1024 + (cg % 8) * 16
                off = roff + cg * 16
                s_ = [accs[pl.ds(off, 16)], zero, zero, zero]
                q_ = [accq[pl.ds(off, 16)], zero, zero, zero]
                n = 0
                for wt in range(WD // 8):
                    for ws in range(8):
                        v = buf[pl.ds(cb + wt * 3072 + ws * 128, 16)]
                        s_[n % 4] = s_[n % 4] + v
                        q_[n % 4] = q_[n % 4] + v * v
                        n += 1
                accs[pl.ds(off, 16)] = (s_[0] + s_[1]) + (s_[2] + s_[3])
                accq[pl.ds(off, 16)] = (q_[0] + q_[1]) + (q_[2] + q_[3])

            plsc.parallel_loop(0, GPB, 1, unroll=2)(cgbody)

            @pl.when(kc + 3 < NCHUNK)
            def _():
                start(kc + 3, buf, sem)

        start(0, buf0, sem0)
        start(1, buf1, sem1)
        start(2, buf2, sem2)

        bufs = (buf0, buf1, buf2)
        sems = (sem0, sem1, sem2)

        def chunk_trio(i, carry):
            for j in range(3):
                reduce_chunk(3 * i + j, bufs[j], sems[j])
            return carry

        lax.fori_loop(0, (NCHUNK - 1) // 3, chunk_trio, 0)
        reduce_chunk(NCHUNK - 1, buf0, sem0)

        def statbody(g, carry):
            s = accs[pl.ds(g * 16, 16)]
            s2 = accq[pl.ds(g * 16, 16)]
            mean = s * (1.0 / HW)
            var = (s2 - s * s * (1.0 / HW)) * (1.0 / (HW - 1))
            vc = jnp.maximum(var, 1e-30)
            std = vc * _rsqrt_nw(vc)
            mean_vm[pl.ds(g * 16, 16)] = _bf16_rne(mean)
            std_vm[pl.ds(g * 16, 16)] = _bf16_rne(std)
            return carry

        lax.fori_loop(0, GROUPS, statbody, 0)

        usage = jnp.zeros((16,), f32)
        for bl in range(BATCHES_PER_SUB):
            def p2body(kg, lins):
                base = (bl * GPB + kg) * 16
                mv = mean_vm[pl.ds(base, 16)]
                sv = std_vm[pl.ds(base, 16)]
                lins = list(lins)
                for l in range(16):
                    cc = kg * 16 + l
                    wm = wt_vm[pl.ds(cc * E, E)]
                    ws = wt_vm[pl.ds((C + cc) * E, E)]
                    lins[l % 4] = lins[l % 4] + mv[l] * wm + sv[l] * ws
                return tuple(lins)

            lins = plsc.parallel_loop(0, GPB, 1, unroll=2,
                                      carry=(zero,) * 4)(p2body)
            lin = (lins[0] + lins[1]) + (lins[2] + lins[3])

            m1 = jnp.max(lin)
            e1 = jnp.exp(lin - m1)
            p1 = e1 * _recip_nw(zero + jnp.sum(e1))
            lg = jnp.clip(p1, -30.0, 30.0)
            m2 = jnp.max(lg)
            e2 = jnp.exp(lg - m2)
            p2 = e2 * _recip_nw(zero + jnp.sum(e2))
            v1 = jnp.max(p2)
            i1 = jnp.min(jnp.where(p2 == v1, iota, E))
            neg = jnp.where(iota == i1, -1e30, p2)
            v2 = jnp.max(neg)
            i2 = jnp.min(jnp.where(neg == v2, iota, E))
            den = v1 + v2 + 1e-6
            wnum = jnp.where(iota == 0, v1, jnp.where(iota == 1, v2, 0.0))
            wvec = wnum * _recip_nw(zero + den)

            ob_probs[pl.ds(bl * E, E)] = p2
            ob_logits[pl.ds(bl * E, E)] = lg
            ob_wts[pl.ds(bl * E, E)] = wvec
            ob_idx[pl.ds(bl * E, E)] = jnp.where(iota == 0, i1,
                                                 jnp.where(iota == 1, i2, 0))
            usage = usage + jnp.where(iota == i1, 1.0, 0.0) \
                          + jnp.where(iota == i2, 1.0, 0.0)

        ob_usage[...] = usage
        obase = wid * BATCHES_PER_SUB * E
        nout = BATCHES_PER_SUB * E
        pltpu.sync_copy(ob_probs, probs_o.at[pl.ds(obase, nout)])
        pltpu.sync_copy(ob_logits, logits_o.at[pl.ds(obase, nout)])
        pltpu.sync_copy(ob_wts, wts_o.at[pl.ds(obase, nout)])
        pltpu.sync_copy(ob_idx, idx_o.at[pl.ds(obase, nout)])
        pltpu.sync_copy(ob_usage, usage_o.at[pl.ds(wid * E, E)])

    return sck(x_flat, wt_flat)


@jax.jit
def kernel(x, W):
    xp = x.transpose(0, 2, 3, 1).reshape(B, H, WD // 8, 8, C // 128, 128)
    x_flat = xp.transpose(0, 1, 2, 4, 3, 5).reshape(-1)
    wt_flat = W.T.astype(jnp.bfloat16).astype(jnp.float32).reshape(-1)
    probs, logits, wts, idxo, usage = _sc_router(x_flat, wt_flat)
    probs = probs.reshape(B, E)
    logits = logits.reshape(B, E)
    ti = idxo.reshape(B, E)[:, :2]
    routing_weights = wts.reshape(B, E)[:, :2].reshape(B, 2, 1, 1)
    routing_indices = ti.reshape(B, 2, 1, 1)
    expert_usage = jnp.sum(usage.reshape(NSUB, E), axis=0) * (1.0 / (B * 2))
    return (routing_weights, routing_indices, probs, logits, ti, expert_usage)


# speedup vs baseline: 1.7102x; 1.0465x over previous
import functools

import jax
import jax.numpy as jnp
from jax import lax
from jax.experimental import pallas as pl
from jax.experimental.pallas import tpu as pltpu
from jax.experimental.pallas import tpu_sc as plsc

B = 64
C = 384
H = 32
WD = 32
HW = 1024
E = 16
NSUB = 32
BATCHES_PER_SUB = B // NSUB
SLAB = WD * C
NCHUNK = BATCHES_PER_SUB * H
GROUPS = BATCHES_PER_SUB * C // 16
GPB = C // 16


def _bf16_rne(v):
    u = lax.bitcast_convert_type(v, jnp.uint32)
    u = (u + jnp.uint32(0x7FFF) + ((u >> jnp.uint32(16)) & jnp.uint32(1))) \
        & jnp.uint32(0xFFFF0000)
    return lax.bitcast_convert_type(u, jnp.float32)


def _recip_nw(d):
    r = 1.0 / d
    r = r * (2.0 - d * r)
    r = r * (2.0 - d * r)
    return r


def _rsqrt_nw(v):
    i = lax.bitcast_convert_type(v, jnp.int32)
    i = jnp.int32(0x5F3759DF) - (i >> 1)
    y = lax.bitcast_convert_type(i, jnp.float32)
    for _ in range(4):
        y = y * (1.5 - 0.5 * v * y * y)
    return y


def _sc_router(x_flat, wt_flat):
    mesh = plsc.VectorSubcoreMesh(core_axis_name="c", subcore_axis_name="s")
    f32 = jnp.float32
    i32 = jnp.int32

    out_type = (
        jax.ShapeDtypeStruct((B * E,), f32),
        jax.ShapeDtypeStruct((B * E,), f32),
        jax.ShapeDtypeStruct((B * E,), f32),
        jax.ShapeDtypeStruct((B * E,), i32),
        jax.ShapeDtypeStruct((NSUB * E,), f32),
    )
    scratch = [
        pltpu.VMEM((SLAB,), f32),
        pltpu.VMEM((SLAB,), f32),
        pltpu.VMEM((SLAB,), f32),
        pltpu.VMEM((SLAB,), f32),
        pltpu.VMEM((2 * C * E,), f32),
        pltpu.VMEM((GROUPS * 16,), f32),
        pltpu.VMEM((GROUPS * 16,), f32),
        pltpu.VMEM((BATCHES_PER_SUB * E,), f32),
        pltpu.VMEM((BATCHES_PER_SUB * E,), f32),
        pltpu.VMEM((BATCHES_PER_SUB * E,), f32),
        pltpu.VMEM((BATCHES_PER_SUB * E,), i32),
        pltpu.VMEM((E,), f32),
        pltpu.SemaphoreType.DMA,
        pltpu.SemaphoreType.DMA,
        pltpu.SemaphoreType.DMA,
        pltpu.SemaphoreType.DMA,
    ]

    @functools.partial(pl.kernel, out_type=out_type, mesh=mesh,
                       scratch_types=scratch,
                       compiler_params=pltpu.CompilerParams(
                           needs_layout_passes=False))
    def sck(x_hbm, wt_hbm, probs_o, logits_o, wts_o, idx_o, usage_o,
            buf0, buf1, buf2, buf3, wt_vm, accs, accq,
            ob_probs, ob_logits, ob_wts, ob_idx, ob_usage,
            sem0, sem1, sem2, sem3):
        wid = lax.axis_index("s") * 2 + lax.axis_index("c")
        base_elem = wid * NCHUNK * SLAB

        pltpu.sync_copy(wt_hbm, wt_vm)

        iota = lax.iota(i32, 16)
        zero = jnp.zeros((16,), f32)

        def start(k, buf, sem):
            src = x_hbm.at[pl.ds(base_elem + k * SLAB, SLAB)]
            pltpu.async_copy(src, buf, sem)

        def wait(k, buf, sem):
            src = x_hbm.at[pl.ds(base_elem + k * SLAB, SLAB)]
            pltpu.make_async_copy(src, buf, sem).wait()

        def zbody(g, carry):
            accs[pl.ds(g * 16, 16)] = zero
            accq[pl.ds(g * 16, 16)] = zero
            return carry

        lax.fori_loop(0, GROUPS, zbody, 0)

        def reduce_chunk(kc, buf, sem):
            wait(kc, buf, sem)
            roff = (kc // H) * C

            def cgbody(cg):
                cb = (cg // 8) * 1024 + (cg % 8) * 16
                off = roff + cg * 16
                s_ = [accs[pl.ds(off, 16)], zero, zero, zero]
                q_ = [accq[pl.ds(off, 16)], zero, zero, zero]
                n = 0
                for wt in range(WD // 8):
                    for ws in range(8):
                        v = buf[pl.ds(cb + wt * 3072 + ws * 128, 16)]
                        s_[n % 4] = s_[n % 4] + v
                        q_[n % 4] = q_[n % 4] + v * v
                        n += 1
                accs[pl.ds(off, 16)] = (s_[0] + s_[1]) + (s_[2] + s_[3])
                accq[pl.ds(off, 16)] = (q_[0] + q_[1]) + (q_[2] + q_[3])

            plsc.parallel_loop(0, GPB, 1, unroll=2)(cgbody)

            @pl.when(kc + 4 < NCHUNK)
            def _():
                start(kc + 4, buf, sem)

        start(0, buf0, sem0)
        start(1, buf1, sem1)
        start(2, buf2, sem2)
        start(3, buf3, sem3)

        bufs = (buf0, buf1, buf2, buf3)
        sems = (sem0, sem1, sem2, sem3)

        def chunk_quad(i, carry):
            for j in range(4):
                reduce_chunk(4 * i + j, bufs[j], sems[j])
            return carry

        lax.fori_loop(0, NCHUNK // 4, chunk_quad, 0)

        def statbody(g, carry):
            s = accs[pl.ds(g * 16, 16)]
            s2 = accq[pl.ds(g * 16, 16)]
            mean = s * (1.0 / HW)
            var = (s2 - s * s * (1.0 / HW)) * (1.0 / (HW - 1))
            vc = jnp.maximum(var, 1e-30)
            std = vc * _rsqrt_nw(vc)
            accs[pl.ds(g * 16, 16)] = _bf16_rne(mean)
            accq[pl.ds(g * 16, 16)] = _bf16_rne(std)
            return carry

        lax.fori_loop(0, GROUPS, statbody, 0)

        usage = jnp.zeros((16,), f32)
        for bl in range(BATCHES_PER_SUB):
            def p2body(kg, lins):
                base = (bl * GPB + kg) * 16
                mv = accs[pl.ds(base, 16)]
                sv = accq[pl.ds(base, 16)]
                lins = list(lins)
                for l in range(16):
                    cc = kg * 16 + l
                    wm = wt_vm[pl.ds(cc * E, E)]
                    ws = wt_vm[pl.ds((C + cc) * E, E)]
                    lins[l % 4] = lins[l % 4] + mv[l] * wm + sv[l] * ws
                return tuple(lins)

            lins = plsc.parallel_loop(0, GPB, 1, unroll=2,
                                      carry=(zero,) * 4)(p2body)
            lin = (lins[0] + lins[1]) + (lins[2] + lins[3])

            m1 = jnp.max(lin)
            e1 = jnp.exp(lin - m1)
            p1 = e1 * _recip_nw(zero + jnp.sum(e1))
            lg = jnp.clip(p1, -30.0, 30.0)
            m2 = jnp.max(lg)
            e2 = jnp.exp(lg - m2)
            p2 = e2 * _recip_nw(zero + jnp.sum(e2))
            v1 = jnp.max(p2)
            i1 = jnp.min(jnp.where(p2 == v1, iota, E))
            neg = jnp.where(iota == i1, -1e30, p2)
            v2 = jnp.max(neg)
            i2 = jnp.min(jnp.where(neg == v2, iota, E))
            den = v1 + v2 + 1e-6
            wnum = jnp.where(iota == 0, v1, jnp.where(iota == 1, v2, 0.0))
            wvec = wnum * _recip_nw(zero + den)

            ob_probs[pl.ds(bl * E, E)] = p2
            ob_logits[pl.ds(bl * E, E)] = lg
            ob_wts[pl.ds(bl * E, E)] = wvec
            ob_idx[pl.ds(bl * E, E)] = jnp.where(iota == 0, i1,
                                                 jnp.where(iota == 1, i2, 0))
            usage = usage + jnp.where(iota == i1, 1.0, 0.0) \
                          + jnp.where(iota == i2, 1.0, 0.0)

        ob_usage[...] = usage
        obase = wid * BATCHES_PER_SUB * E
        nout = BATCHES_PER_SUB * E
        pltpu.sync_copy(ob_probs, probs_o.at[pl.ds(obase, nout)])
        pltpu.sync_copy(ob_logits, logits_o.at[pl.ds(obase, nout)])
        pltpu.sync_copy(ob_wts, wts_o.at[pl.ds(obase, nout)])
        pltpu.sync_copy(ob_idx, idx_o.at[pl.ds(obase, nout)])
        pltpu.sync_copy(ob_usage, usage_o.at[pl.ds(wid * E, E)])

    return sck(x_flat, wt_flat)


@jax.jit
def kernel(x, W):
    xp = x.transpose(0, 2, 3, 1).reshape(B, H, WD // 8, 8, C // 128, 128)
    x_flat = xp.transpose(0, 1, 2, 4, 3, 5).reshape(-1)
    wt_flat = W.T.astype(jnp.bfloat16).astype(jnp.float32).reshape(-1)
    probs, logits, wts, idxo, usage = _sc_router(x_flat, wt_flat)
    probs = probs.reshape(B, E)
    logits = logits.reshape(B, E)
    ti = idxo.reshape(B, E)[:, :2]
    routing_weights = wts.reshape(B, E)[:, :2].reshape(B, 2, 1, 1)
    routing_indices = ti.reshape(B, 2, 1, 1)
    expert_usage = jnp.sum(usage.reshape(NSUB, E), axis=0) * (1.0 / (B * 2))
    return (routing_weights, routing_indices, probs, logits, ti, expert_usage)
